# Initial kernel scaffold; baseline (speedup 1.0000x reference)
import functools

import jax
import jax.numpy as jnp
from jax import lax
from jax.experimental import pallas as pl
from jax.experimental.pallas import tpu as pltpu
from jax.experimental.pallas import tpu_sc as plsc

B = 16384
S = 100
FID = 1000000

NW = 32
LANE = 128
ROWS = (B * S) // LANE
RPW = ROWS // NW
CHUNK = 80
NCHUNK = RPW // CHUNK

_sc_mesh = plsc.VectorSubcoreMesh(core_axis_name="c", subcore_axis_name="s")


@functools.partial(
    pl.kernel,
    out_type=jax.ShapeDtypeStruct((ROWS, LANE, 2), jnp.float32),
    mesh=_sc_mesh,
    scratch_types=[
        pltpu.VMEM((RPW, LANE), jnp.int32),
        pltpu.VMEM((CHUNK, LANE, 2), jnp.float32),
        pltpu.SemaphoreType.DMA,
    ],
)
def _sc_gather(idx_hbm, tab_hbm, out_hbm, idx_v, vals_v, sem):
    w = lax.axis_index("s") * 2 + lax.axis_index("c")
    base = w * RPW
    pltpu.sync_copy(idx_hbm.at[pl.ds(base, RPW)], idx_v)

    def chunk_body(ci, carry):
        row0 = ci * CHUNK

        def fire(j, c):
            pltpu.async_copy(tab_hbm.at[idx_v.at[row0 + j]], vals_v.at[j], sem)
            return c

        lax.fori_loop(0, CHUNK, fire, 0)
        pltpu.make_async_copy(
            out_hbm.at[pl.ds(base + row0, CHUNK)], vals_v, sem
        ).wait()
        pltpu.sync_copy(vals_v, out_hbm.at[pl.ds(base + row0, CHUNK)])
        return carry

    lax.fori_loop(0, NCHUNK, chunk_body, 0)


BM = 2048
NB = B // BM


def _tc_tower(x_ref, lab_ref, w1_ref, b1_ref, w2_ref, b2_ref, w3_ref,
              gb_ref, pred_ref, loss_ref, acc_ref):
    i = pl.program_id(0)
    x = x_ref[...]
    col = lax.broadcasted_iota(jnp.int32, (BM, 2 * S), 1)
    even = (col % 2) == 0
    bias_sum = jnp.sum(jnp.where(even, x, 0.0), axis=1)
    cert_sum = jnp.sum(jnp.where(even, 0.0, x), axis=1)

    h = jnp.dot(x, w1_ref[...], preferred_element_type=jnp.float32)
    h = jnp.maximum(h + b1_ref[...], 0.0)
    h = jnp.dot(h, w2_ref[...], preferred_element_type=jnp.float32)
    h = jnp.maximum(h + b2_ref[...], 0.0)
    nn_out = jnp.sum(h * w3_ref[...], axis=1)

    logits = bias_sum + gb_ref[0] + nn_out
    pred_ref[0, :] = jax.nn.sigmoid(logits)

    raw = jax.nn.sigmoid(cert_sum) + 0.5
    lab = lab_ref[0, :]
    xent = (jnp.maximum(logits, 0.0) - logits * lab
            + jnp.log1p(jnp.exp(-jnp.abs(logits))))
    pa = jnp.sum(xent * raw)
    pc = jnp.sum(raw)

    @pl.when(i == 0)
    def _init():
        acc_ref[0] = pa
        acc_ref[1] = pc

    @pl.when(i > 0)
    def _accum():
        acc_ref[0] += pa
        acc_ref[1] += pc

    @pl.when(i == NB - 1)
    def _fin():
        loss_ref[0] = acc_ref[0] * jnp.float32(B) / acc_ref[1]


_tower_call = pl.pallas_call(
    _tc_tower,
    grid=(NB,),
    in_specs=[
        pl.BlockSpec((BM, 2 * S), lambda i: (i, 0)),
        pl.BlockSpec((1, BM), lambda i: (i, 0)),
        pl.BlockSpec((2 * S, 512), lambda i: (0, 0)),
        pl.BlockSpec((1, 512), lambda i: (0, 0)),
        pl.BlockSpec((512, 256), lambda i: (0, 0)),
        pl.BlockSpec((1, 256), lambda i: (0, 0)),
        pl.BlockSpec((1, 256), lambda i: (0, 0)),
        pl.BlockSpec(memory_space=pltpu.SMEM),
    ],
    out_specs=[
        pl.BlockSpec((1, BM), lambda i: (i, 0)),
        pl.BlockSpec(memory_space=pltpu.SMEM),
    ],
    out_shape=[
        jax.ShapeDtypeStruct((NB, BM), jnp.float32),
        jax.ShapeDtypeStruct((1,), jnp.float32),
    ],
    scratch_shapes=[pltpu.SMEM((2,), jnp.float32)],
)


def kernel(slot_bias_fid_index, label, sparse_bias, certain_bias,
           global_bias, W1, b1, W2, b2, W3, b3):
    tab = jnp.stack([sparse_bias, certain_bias], axis=-1)
    idx = slot_bias_fid_index.reshape(ROWS, LANE)

    packed = _sc_gather(idx, tab)
    x = packed.reshape(B, 2 * S)

    W1p = jnp.stack([W1, jnp.zeros_like(W1)], axis=1).reshape(2 * S, 512)
    gb = (global_bias[0] + b3[0]).reshape(1)

    pred, loss = _tower_call(
        x, label.reshape(NB, BM), W1p, b1.reshape(1, 512), W2,
        b2.reshape(1, 256), W3.reshape(1, 256), gb)
    return pred.reshape(B), loss[0]


# speedup vs baseline: 1.5882x; 1.5882x over previous
import functools

import jax
import jax.numpy as jnp
from jax import lax
from jax.experimental import pallas as pl
from jax.experimental.pallas import tpu as pltpu
from jax.experimental.pallas import tpu_sc as plsc

B = 16384
S = 100
FID = 1000000

NW = 32
LANE = 128
ROWS = (B * S) // LANE
RPW = ROWS // NW
CHUNK = 100
NCHUNK = RPW // CHUNK

_sc_mesh = plsc.VectorSubcoreMesh(core_axis_name="c", subcore_axis_name="s")


@functools.partial(
    pl.kernel,
    out_type=jax.ShapeDtypeStruct((ROWS, LANE), jnp.int32),
    mesh=_sc_mesh,
    scratch_types=[
        pltpu.VMEM((RPW, LANE), jnp.int32),
        pltpu.VMEM((CHUNK, LANE), jnp.int32),
        pltpu.SemaphoreType.DMA,
    ],
    compiler_params=pltpu.CompilerParams(use_tc_tiling_on_sc=False),
)
def _sc_gather(idx_hbm, tab_hbm, out_hbm, idx_v, vals_v, sem):
    w = lax.axis_index("s") * 2 + lax.axis_index("c")
    base = w * RPW
    pltpu.sync_copy(idx_hbm.at[pl.ds(base, RPW)], idx_v)

    def chunk_body(ci, carry):
        row0 = ci * CHUNK

        def fire(j, c):
            pltpu.async_copy(tab_hbm.at[idx_v.at[row0 + j]], vals_v.at[j], sem)
            return c

        lax.fori_loop(0, CHUNK, fire, 0)
        pltpu.make_async_copy(
            out_hbm.at[pl.ds(base + row0, CHUNK)], vals_v, sem
        ).wait()
        pltpu.sync_copy(vals_v, out_hbm.at[pl.ds(base + row0, CHUNK)])
        return carry

    lax.fori_loop(0, NCHUNK, chunk_body, 0)


BM = 2048
NB = B // BM


def _tc_tower(x_ref, lab_ref, w1_ref, b1_ref, w2_ref, b2_ref, w3_ref,
              gb_ref, pred_ref, loss_ref, acc_ref):
    i = pl.program_id(0)
    xi = x_ref[...]
    x_sp = lax.bitcast_convert_type(xi & jnp.int32(-65536), jnp.float32)
    x_ct = lax.bitcast_convert_type(xi << 16, jnp.float32)
    bias_sum = jnp.sum(x_sp, axis=1)
    cert_sum = jnp.sum(x_ct, axis=1)

    h = jnp.dot(x_sp.astype(jnp.bfloat16), w1_ref[...],
                preferred_element_type=jnp.float32)
    h = jnp.maximum(h + b1_ref[...], 0.0)
    h = jnp.dot(h.astype(jnp.bfloat16), w2_ref[...],
                preferred_element_type=jnp.float32)
    h = jnp.maximum(h + b2_ref[...], 0.0)
    nn_out = jnp.sum(h * w3_ref[...], axis=1)

    logits = bias_sum + gb_ref[0] + nn_out
    pred_ref[0, 0, :] = jax.nn.sigmoid(logits)

    raw = jax.nn.sigmoid(cert_sum) + 0.5
    lab = lab_ref[0, 0, :]
    xent = (jnp.maximum(logits, 0.0) - logits * lab
            + jnp.log1p(jnp.exp(-jnp.abs(logits))))
    pa = jnp.sum(xent * raw)
    pc = jnp.sum(raw)

    @pl.when(i == 0)
    def _init():
        acc_ref[0] = pa
        acc_ref[1] = pc

    @pl.when(i > 0)
    def _accum():
        acc_ref[0] += pa
        acc_ref[1] += pc

    @pl.when(i == NB - 1)
    def _fin():
        loss_ref[0] = acc_ref[0] * jnp.float32(B) / acc_ref[1]


_tower_call = pl.pallas_call(
    _tc_tower,
    grid=(NB,),
    in_specs=[
        pl.BlockSpec((BM, S), lambda i: (i, 0)),
        pl.BlockSpec((1, 1, BM), lambda i: (i, 0, 0)),
        pl.BlockSpec((S, 512), lambda i: (0, 0)),
        pl.BlockSpec((1, 512), lambda i: (0, 0)),
        pl.BlockSpec((512, 256), lambda i: (0, 0)),
        pl.BlockSpec((1, 256), lambda i: (0, 0)),
        pl.BlockSpec((1, 256), lambda i: (0, 0)),
        pl.BlockSpec(memory_space=pltpu.SMEM),
    ],
    out_specs=[
        pl.BlockSpec((1, 1, BM), lambda i: (i, 0, 0)),
        pl.BlockSpec(memory_space=pltpu.SMEM),
    ],
    out_shape=[
        jax.ShapeDtypeStruct((NB, 1, BM), jnp.float32),
        jax.ShapeDtypeStruct((1,), jnp.float32),
    ],
    scratch_shapes=[pltpu.SMEM((2,), jnp.float32)],
)


def kernel(slot_bias_fid_index, label, sparse_bias, certain_bias,
           global_bias, W1, b1, W2, b2, W3, b3):
    sb = lax.bitcast_convert_type(
        sparse_bias.astype(jnp.bfloat16), jnp.uint16).astype(jnp.uint32)
    cb = lax.bitcast_convert_type(
        certain_bias.astype(jnp.bfloat16), jnp.uint16).astype(jnp.uint32)
    tab = lax.bitcast_convert_type((sb << 16) | cb, jnp.int32)

    idx = slot_bias_fid_index.reshape(ROWS, LANE)
    packed = _sc_gather(idx, tab)
    x = packed.reshape(B, S)

    gb = (global_bias[0] + b3[0]).reshape(1)
    pred, loss = _tower_call(
        x, label.reshape(NB, 1, BM), W1.astype(jnp.bfloat16),
        b1.reshape(1, 512), W2.astype(jnp.bfloat16), b2.reshape(1, 256),
        W3.reshape(1, 256), gb)
    return pred.reshape(B), loss[0]


# speedup vs baseline: 2.0707x; 1.3038x over previous
import functools

import jax
import jax.numpy as jnp
from jax import lax
from jax.experimental import pallas as pl
from jax.experimental.pallas import tpu as pltpu
from jax.experimental.pallas import tpu_sc as plsc

B = 16384
S = 100
FID = 1000000

NW = 32
RPW = B // NW
CHUNK = 128
NCHUNK = RPW // CHUNK

_sc_mesh = plsc.VectorSubcoreMesh(core_axis_name="c", subcore_axis_name="s")


@functools.partial(
    pl.kernel,
    out_type=jax.ShapeDtypeStruct((B, S), jnp.int32),
    mesh=_sc_mesh,
    scratch_types=[
        pltpu.VMEM((RPW, S), jnp.int32),
        pltpu.VMEM((CHUNK, S), jnp.int32),
        pltpu.SemaphoreType.DMA,
    ],
    compiler_params=pltpu.CompilerParams(use_tc_tiling_on_sc=True),
)
def _sc_gather(idx_hbm, tab_hbm, out_hbm, idx_v, vals_v, sem):
    w = lax.axis_index("s") * 2 + lax.axis_index("c")
    base = w * RPW
    pltpu.sync_copy(idx_hbm.at[pl.ds(base, RPW)], idx_v)

    def chunk_body(ci, carry):
        row0 = ci * CHUNK

        def fire(j, c):
            pltpu.async_copy(tab_hbm.at[idx_v.at[row0 + j]], vals_v.at[j], sem)
            return c

        lax.fori_loop(0, CHUNK, fire, 0)

        def drain(j, c):
            pltpu.make_async_copy(
                tab_hbm.at[idx_v.at[row0 + j]], vals_v.at[j], sem
            ).wait()
            return c

        lax.fori_loop(0, CHUNK, drain, 0)
        pltpu.sync_copy(vals_v, out_hbm.at[pl.ds(base + row0, CHUNK)])
        return carry

    lax.fori_loop(0, NCHUNK, chunk_body, 0)


BM = 2048
NB = B // BM
N1 = 640


def _tc_tower(x_ref, lab_ref, w1_ref, b1_ref, cc_ref, w2_ref, b2_ref,
              w3_ref, gb_ref, pred_ref, loss_ref, acc_ref):
    i = pl.program_id(0)
    xi = x_ref[...]
    x_sp = lax.bitcast_convert_type(
        xi & jnp.int32(-65536), jnp.float32).astype(jnp.bfloat16)
    x_ct = lax.bitcast_convert_type(
        xi << 16, jnp.float32).astype(jnp.bfloat16)

    h0 = jnp.dot(x_sp, w1_ref[...], preferred_element_type=jnp.float32)
    bias_sum = h0[:, 512]
    cp = jnp.dot(x_ct, cc_ref[...], preferred_element_type=jnp.float32)
    cert_sum = cp[:, 0]

    h = jnp.maximum(h0[:, :512] + b1_ref[...], 0.0).astype(jnp.bfloat16)
    h = jnp.dot(h, w2_ref[...], preferred_element_type=jnp.float32)
    h = jnp.maximum(h + b2_ref[...], 0.0).astype(jnp.bfloat16)
    nn = jnp.dot(h, w3_ref[...], preferred_element_type=jnp.float32)
    nn_out = nn[:, 0]

    logits = bias_sum + gb_ref[0] + nn_out
    pred_ref[0, 0, :] = jax.nn.sigmoid(logits)

    raw = jax.nn.sigmoid(cert_sum) + 0.5
    lab = lab_ref[0, 0, :]
    xent = (jnp.maximum(logits, 0.0) - logits * lab
            + jnp.log1p(jnp.exp(-jnp.abs(logits))))
    pa = jnp.sum(xent * raw)
    pc = jnp.sum(raw)

    @pl.when(i == 0)
    def _init():
        acc_ref[0] = pa
        acc_ref[1] = pc

    @pl.when(i > 0)
    def _accum():
        acc_ref[0] += pa
        acc_ref[1] += pc

    @pl.when(i == NB - 1)
    def _fin():
        loss_ref[0] = acc_ref[0] * jnp.float32(B) / acc_ref[1]


_tower_call = pl.pallas_call(
    _tc_tower,
    grid=(NB,),
    in_specs=[
        pl.BlockSpec((BM, S), lambda i: (i, 0)),
        pl.BlockSpec((1, 1, BM), lambda i: (i, 0, 0)),
        pl.BlockSpec((S, N1), lambda i: (0, 0)),
        pl.BlockSpec((1, 512), lambda i: (0, 0)),
        pl.BlockSpec((S, 128), lambda i: (0, 0)),
        pl.BlockSpec((512, 256), lambda i: (0, 0)),
        pl.BlockSpec((1, 256), lambda i: (0, 0)),
        pl.BlockSpec((256, 128), lambda i: (0, 0)),
        pl.BlockSpec(memory_space=pltpu.SMEM),
    ],
    out_specs=[
        pl.BlockSpec((1, 1, BM), lambda i: (i, 0, 0)),
        pl.BlockSpec(memory_space=pltpu.SMEM),
    ],
    out_shape=[
        jax.ShapeDtypeStruct((NB, 1, BM), jnp.float32),
        jax.ShapeDtypeStruct((1,), jnp.float32),
    ],
    scratch_shapes=[pltpu.SMEM((2,), jnp.float32)],
)


def kernel(slot_bias_fid_index, label, sparse_bias, certain_bias,
           global_bias, W1, b1, W2, b2, W3, b3):
    sb = lax.bitcast_convert_type(
        sparse_bias.astype(jnp.bfloat16), jnp.uint16).astype(jnp.uint32)
    cb = lax.bitcast_convert_type(
        certain_bias.astype(jnp.bfloat16), jnp.uint16).astype(jnp.uint32)
    tab = lax.bitcast_convert_type((sb << 16) | cb, jnp.int32)

    x = _sc_gather(slot_bias_fid_index, tab)

    bf = jnp.bfloat16
    w1a = jnp.zeros((S, N1), bf).at[:, :512].set(W1.astype(bf))
    w1a = w1a.at[:, 512].set(jnp.float32(1.0).astype(bf))
    cc = jnp.zeros((S, 128), bf).at[:, 0].set(jnp.float32(1.0).astype(bf))
    w3c = jnp.zeros((256, 128), bf).at[:, 0].set(W3[:, 0].astype(bf))
    gb = (global_bias[0] + b3[0]).reshape(1)

    pred, loss = _tower_call(
        x, label.reshape(NB, 1, BM), w1a, b1.reshape(1, 512), cc,
        W2.astype(bf), b2.reshape(1, 256), w3c, gb)
    return pred.reshape(B), loss[0]


# speedup vs baseline: 2.1236x; 1.0256x over previous
import functools

import jax
import jax.numpy as jnp
from jax import lax
from jax.experimental import pallas as pl
from jax.experimental.pallas import tpu as pltpu
from jax.experimental.pallas import tpu_sc as plsc

B = 16384
S = 100
FID = 1000000

NW = 32
RPW = B // NW
CHUNK = 256
NCHUNK = RPW // CHUNK

_sc_mesh = plsc.VectorSubcoreMesh(core_axis_name="c", subcore_axis_name="s")


@functools.partial(
    pl.kernel,
    out_type=jax.ShapeDtypeStruct((B, S), jnp.int32),
    mesh=_sc_mesh,
    scratch_types=[
        pltpu.VMEM((RPW, S), jnp.int32),
        pltpu.VMEM((CHUNK, S), jnp.int32),
        pltpu.SemaphoreType.DMA,
    ],
    compiler_params=pltpu.CompilerParams(use_tc_tiling_on_sc=True),
)
def _sc_gather(idx_hbm, tab_hbm, out_hbm, idx_v, vals_v, sem):
    w = lax.axis_index("s") * 2 + lax.axis_index("c")
    base = w * RPW
    pltpu.sync_copy(idx_hbm.at[pl.ds(base, RPW)], idx_v)

    def chunk_body(ci, carry):
        row0 = ci * CHUNK

        def fire(j, c):
            pltpu.async_copy(tab_hbm.at[idx_v.at[row0 + j]], vals_v.at[j], sem)
            return c

        lax.fori_loop(0, CHUNK, fire, 0)

        def drain(j, c):
            pltpu.make_async_copy(
                tab_hbm.at[idx_v.at[row0 + j]], vals_v.at[j], sem
            ).wait()
            return c

        lax.fori_loop(0, CHUNK, drain, 0)
        pltpu.sync_copy(vals_v, out_hbm.at[pl.ds(base + row0, CHUNK)])
        return carry

    lax.fori_loop(0, NCHUNK, chunk_body, 0)


BM = 4096
NB = B // BM
N1 = 640


def _tc_tower(x_ref, lab_ref, w1_ref, b1_ref, cc_ref, w2_ref, b2_ref,
              w3_ref, gb_ref, pred_ref, loss_ref, acc_ref):
    i = pl.program_id(0)
    xi = x_ref[...]
    x_sp = lax.bitcast_convert_type(
        xi & jnp.int32(-65536), jnp.float32).astype(jnp.bfloat16)
    x_ct = lax.bitcast_convert_type(
        xi << 16, jnp.float32).astype(jnp.bfloat16)

    h0 = jnp.dot(x_sp, w1_ref[...], preferred_element_type=jnp.float32)
    bias_sum = h0[:, 512]
    cp = jnp.dot(x_ct, cc_ref[...], preferred_element_type=jnp.float32)
    cert_sum = cp[:, 0]

    h = jnp.maximum(h0[:, :512] + b1_ref[...], 0.0).astype(jnp.bfloat16)
    h = jnp.dot(h, w2_ref[...], preferred_element_type=jnp.float32)
    h = jnp.maximum(h + b2_ref[...], 0.0).astype(jnp.bfloat16)
    nn = jnp.dot(h, w3_ref[...], preferred_element_type=jnp.float32)
    nn_out = nn[:, 0]

    logits = bias_sum + gb_ref[0] + nn_out
    pred_ref[0, 0, :] = jax.nn.sigmoid(logits)

    raw = jax.nn.sigmoid(cert_sum) + 0.5
    lab = lab_ref[0, 0, :]
    xent = (jnp.maximum(logits, 0.0) - logits * lab
            + jnp.log1p(jnp.exp(-jnp.abs(logits))))
    pa = jnp.sum(xent * raw)
    pc = jnp.sum(raw)

    @pl.when(i == 0)
    def _init():
        acc_ref[0] = pa
        acc_ref[1] = pc

    @pl.when(i > 0)
    def _accum():
        acc_ref[0] += pa
        acc_ref[1] += pc

    @pl.when(i == NB - 1)
    def _fin():
        loss_ref[0] = acc_ref[0] * jnp.float32(B) / acc_ref[1]


_tower_call = pl.pallas_call(
    _tc_tower,
    grid=(NB,),
    in_specs=[
        pl.BlockSpec((BM, S), lambda i: (i, 0)),
        pl.BlockSpec((1, 1, BM), lambda i: (i, 0, 0)),
        pl.BlockSpec((S, N1), lambda i: (0, 0)),
        pl.BlockSpec((1, 512), lambda i: (0, 0)),
        pl.BlockSpec((S, 128), lambda i: (0, 0)),
        pl.BlockSpec((512, 256), lambda i: (0, 0)),
        pl.BlockSpec((1, 256), lambda i: (0, 0)),
        pl.BlockSpec((256, 128), lambda i: (0, 0)),
        pl.BlockSpec(memory_space=pltpu.SMEM),
    ],
    out_specs=[
        pl.BlockSpec((1, 1, BM), lambda i: (i, 0, 0)),
        pl.BlockSpec(memory_space=pltpu.SMEM),
    ],
    out_shape=[
        jax.ShapeDtypeStruct((NB, 1, BM), jnp.float32),
        jax.ShapeDtypeStruct((1,), jnp.float32),
    ],
    scratch_shapes=[pltpu.SMEM((2,), jnp.float32)],
)


def kernel(slot_bias_fid_index, label, sparse_bias, certain_bias,
           global_bias, W1, b1, W2, b2, W3, b3):
    sb = lax.bitcast_convert_type(
        sparse_bias.astype(jnp.bfloat16), jnp.uint16).astype(jnp.uint32)
    cb = lax.bitcast_convert_type(
        certain_bias.astype(jnp.bfloat16), jnp.uint16).astype(jnp.uint32)
    tab = lax.bitcast_convert_type((sb << 16) | cb, jnp.int32)

    x = _sc_gather(slot_bias_fid_index, tab)

    bf = jnp.bfloat16
    w1a = jnp.zeros((S, N1), bf).at[:, :512].set(W1.astype(bf))
    w1a = w1a.at[:, 512].set(jnp.float32(1.0).astype(bf))
    cc = jnp.zeros((S, 128), bf).at[:, 0].set(jnp.float32(1.0).astype(bf))
    w3c = jnp.zeros((256, 128), bf).at[:, 0].set(W3[:, 0].astype(bf))
    gb = (global_bias[0] + b3[0]).reshape(1)

    pred, loss = _tower_call(
        x, label.reshape(NB, 1, BM), w1a, b1.reshape(1, 512), cc,
        W2.astype(bf), b2.reshape(1, 256), w3c, gb)
    return pred.reshape(B), loss[0]
